# baseline (speedup 1.0000x reference)
import functools

import jax
import jax.numpy as jnp
from jax import lax
from jax.experimental import pallas as pl
from jax.experimental.pallas import tpu as pltpu
from jax.experimental.pallas import tpu_sc as plsc

N = 100000
K = 64
B = 1024
H = 49920
H2 = N - H

NC = 2
NS = 16
L = 16
NW = NC * NS
BPW = B // NW
RPW = 2 * K // NW

_mesh = plsc.VectorSubcoreMesh(core_axis_name="c", subcore_axis_name="s")


@functools.partial(
    pl.kernel,
    out_type=[
        jax.ShapeDtypeStruct((K, B), jnp.float32),
        jax.ShapeDtypeStruct((K, B), jnp.float32),
        jax.ShapeDtypeStruct((B,), jnp.float32),
        jax.ShapeDtypeStruct((B,), jnp.float32),
    ],
    mesh=_mesh,
    compiler_params=pltpu.CompilerParams(needs_layout_passes=False),
    scratch_types=[
        pltpu.VMEM((B,), jnp.int32),
        pltpu.VMEM((H,), jnp.float32),
        pltpu.VMEM((H2,), jnp.float32),
        pltpu.VMEM((B,), jnp.float32),
        pltpu.VMEM((BPW,), jnp.float32),
        pltpu.VMEM((BPW,), jnp.float32),
        pltpu.SemaphoreType.DMA,
        pltpu.SemaphoreType.DMA,
        pltpu.SemaphoreType.DMA,
    ],
)
def _sc_encoder(idx_hbm, wpm_hbm, wpl_hbm, whm_hbm, whl_hbm,
                pm_hbm, pv_hbm, hm_hbm, hv_hbm,
                idx_v, rowa_v, rowb_v, ext_v, hm_v, hv_v,
                sema, semb, semh):
    wid = lax.axis_index("s") * NC + lax.axis_index("c")
    base_b = wid * BPW

    pltpu.sync_copy(idx_hbm, idx_v)

    hcps = [
        pltpu.async_copy(whm_hbm.at[idx_v.at[pl.ds(base_b, BPW)]], hm_v, semh),
        pltpu.async_copy(whl_hbm.at[idx_v.at[pl.ds(base_b, BPW)]], hv_v, semh),
    ]

    srcs = (wpm_hbm, wpm_hbm, wpl_hbm, wpl_hbm)
    dsts = (pm_hbm, pm_hbm, pv_hbm, pv_hbm)

    def row_of(t):
        return wid + (t % 2) * NW

    def stream(t, half, sem):
        if half == 0:
            return pltpu.async_copy(
                srcs[t].at[row_of(t)].at[pl.ds(0, H)], rowa_v, sema)
        return pltpu.async_copy(
            srcs[t].at[row_of(t)].at[pl.ds(H, H2)], rowb_v, semb)

    cpa = stream(0, 0, sema)
    cpb = stream(0, 1, semb)
    wcps = []
    for t in range(RPW):
        cpa.wait()
        for j in range(B // L):
            iv = idx_v[pl.ds(j * L, L)]
            ia = jnp.minimum(iv, H - 1)
            ext_v[pl.ds(j * L, L)] = plsc.load_gather(rowa_v, [ia])
        if t + 1 < RPW:
            cpa = stream(t + 1, 0, sema)
        cpb.wait()
        for j in range(B // L):
            iv = idx_v[pl.ds(j * L, L)]
            ib = jnp.maximum(iv - H, 0)
            vb = plsc.load_gather(rowb_v, [ib])
            v = jnp.where(iv < H, ext_v[pl.ds(j * L, L)], vb)
            if t >= 2:
                v = jnp.exp(v * 2.0)
            ext_v[pl.ds(j * L, L)] = v
        wcps.append(pltpu.async_copy(ext_v, dsts[t].at[row_of(t)], semh))
        wcps[-1].wait()
        if t + 1 < RPW:
            cpb = stream(t + 1, 1, semb)

    for cp in hcps:
        cp.wait()
    for i in range(BPW // L):
        hv_v[pl.ds(i * L, L)] = jnp.exp(hv_v[pl.ds(i * L, L)] * 2.0)
    pltpu.sync_copy(hm_v, hm_hbm.at[pl.ds(base_b, BPW)])
    pltpu.sync_copy(hv_v, hv_hbm.at[pl.ds(base_b, BPW)])


def kernel(indices, W_pos_mean, W_pos_logvar, W_het_mean, W_het_logvar):
    idx = indices.astype(jnp.int32)
    pm_t, pv_t, hm, hv = _sc_encoder(
        idx,
        W_pos_mean,
        W_pos_logvar,
        W_het_mean.reshape(-1),
        W_het_logvar.reshape(-1),
    )
    return (
        pm_t.T,
        pv_t.T,
        hm.reshape(B, 1),
        hv.reshape(B, 1),
    )


# speedup vs baseline: 1.1076x; 1.1076x over previous
import functools

import jax
import jax.numpy as jnp
from jax import lax
from jax.experimental import pallas as pl
from jax.experimental.pallas import tpu as pltpu
from jax.experimental.pallas import tpu_sc as plsc

N = 100000
K = 64
B = 1024

NC = 2
NS = 16
L = 16
NW = NC * NS
BPW = B // NW
RPW = 2 * K // NW

_mesh = plsc.VectorSubcoreMesh(core_axis_name="c", subcore_axis_name="s")


@functools.partial(
    pl.kernel,
    out_type=[
        jax.ShapeDtypeStruct((K, B), jnp.float32),
        jax.ShapeDtypeStruct((K, B), jnp.float32),
        jax.ShapeDtypeStruct((B,), jnp.float32),
        jax.ShapeDtypeStruct((B,), jnp.float32),
    ],
    mesh=_mesh,
    compiler_params=pltpu.CompilerParams(needs_layout_passes=False),
    scratch_types=[
        pltpu.VMEM((B,), jnp.int32),
        pltpu.VMEM((N,), jnp.float32),
        pltpu.VMEM((B,), jnp.float32),
        pltpu.VMEM((B,), jnp.float32),
        pltpu.VMEM((BPW,), jnp.float32),
        pltpu.VMEM((BPW,), jnp.float32),
        pltpu.SemaphoreType.DMA,
        pltpu.SemaphoreType.DMA,
    ],
)
def _sc_encoder(idx_hbm, wpm_hbm, wpl_hbm, whm_hbm, whl_hbm,
                pm_hbm, pv_hbm, hm_hbm, hv_hbm,
                idx_v, row_v, exta_v, extb_v, hm_v, hv_v, sem, semw):
    wid = lax.axis_index("s") * NC + lax.axis_index("c")
    base_b = wid * BPW

    pltpu.sync_copy(idx_hbm, idx_v)

    hcps = [
        pltpu.async_copy(whm_hbm.at[0].at[idx_v.at[pl.ds(base_b, BPW)]],
                         hm_v, sem),
        pltpu.async_copy(whl_hbm.at[0].at[idx_v.at[pl.ds(base_b, BPW)]],
                         hv_v, sem),
    ]

    wcps = []
    for t in range(RPW):
        src = (wpm_hbm, wpm_hbm, wpl_hbm, wpl_hbm)[t]
        dst = (pm_hbm, pm_hbm, pv_hbm, pv_hbm)[t]
        ext_v = (exta_v, extb_v)[t % 2]
        k = wid + (t % 2) * NW
        pltpu.sync_copy(src.at[k], row_v)
        if t < 2:
            for j in range(B // L):
                iv = idx_v[pl.ds(j * L, L)]
                ext_v[pl.ds(j * L, L)] = plsc.load_gather(row_v, [iv])
        else:
            for j in range(B // L):
                iv = idx_v[pl.ds(j * L, L)]
                x = plsc.load_gather(row_v, [iv])
                ext_v[pl.ds(j * L, L)] = jnp.exp(x * 2.0)
        if len(wcps) >= 2:
            wcps[t - 2].wait()
        wcps.append(pltpu.async_copy(ext_v, dst.at[k], semw))

    for cp in hcps:
        cp.wait()
    for i in range(BPW // L):
        hv_v[pl.ds(i * L, L)] = jnp.exp(hv_v[pl.ds(i * L, L)] * 2.0)
    pltpu.sync_copy(hm_v, hm_hbm.at[pl.ds(base_b, BPW)])
    pltpu.sync_copy(hv_v, hv_hbm.at[pl.ds(base_b, BPW)])
    for cp in wcps[-2:]:
        cp.wait()


def kernel(indices, W_pos_mean, W_pos_logvar, W_het_mean, W_het_logvar):
    idx = indices.astype(jnp.int32)
    pm_t, pv_t, hm, hv = _sc_encoder(
        idx, W_pos_mean, W_pos_logvar, W_het_mean, W_het_logvar,
    )
    return (
        pm_t.T,
        pv_t.T,
        hm.reshape(B, 1),
        hv.reshape(B, 1),
    )


# speedup vs baseline: 1.1401x; 1.0294x over previous
import functools

import jax
import jax.numpy as jnp
from jax import lax
from jax.experimental import pallas as pl
from jax.experimental.pallas import tpu as pltpu
from jax.experimental.pallas import tpu_sc as plsc

N = 100000
K = 64
B = 1024

NC = 2
NS = 16
L = 16
NW = NC * NS
BPW = B // NW
RPW = 2 * K // NW

_mesh = plsc.VectorSubcoreMesh(core_axis_name="c", subcore_axis_name="s")


@functools.partial(
    pl.kernel,
    out_type=[
        jax.ShapeDtypeStruct((K, B), jnp.float32),
        jax.ShapeDtypeStruct((K, B), jnp.float32),
        jax.ShapeDtypeStruct((B,), jnp.float32),
        jax.ShapeDtypeStruct((B,), jnp.float32),
    ],
    mesh=_mesh,
    compiler_params=pltpu.CompilerParams(needs_layout_passes=False),
    scratch_types=[
        pltpu.VMEM((B,), jnp.int32),
        pltpu.VMEM((N,), jnp.float32),
        pltpu.VMEM((B,), jnp.float32),
        pltpu.VMEM((B,), jnp.float32),
        pltpu.VMEM((BPW,), jnp.float32),
        pltpu.VMEM((BPW,), jnp.float32),
        pltpu.SemaphoreType.DMA,
        pltpu.SemaphoreType.DMA,
    ],
)
def _sc_encoder(idx_hbm, wpm_hbm, wpl_hbm, whm_hbm, whl_hbm,
                pm_hbm, pv_hbm, hm_hbm, hv_hbm,
                idx_v, row_v, exta_v, extb_v, hm_v, hv_v, sem, semw):
    wid = lax.axis_index("s") * NC + lax.axis_index("c")
    base_b = wid * BPW

    pltpu.sync_copy(idx_hbm, idx_v)

    hcps = [
        pltpu.async_copy(whm_hbm.at[0].at[idx_v.at[pl.ds(base_b, BPW)]],
                         hm_v, sem),
        pltpu.async_copy(whl_hbm.at[0].at[idx_v.at[pl.ds(base_b, BPW)]],
                         hv_v, sem),
    ]

    wcps = []
    for t in range(RPW):
        src = (wpm_hbm, wpm_hbm, wpl_hbm, wpl_hbm)[t]
        dst = (pm_hbm, pm_hbm, pv_hbm, pv_hbm)[t]
        ext_v = (exta_v, extb_v)[t % 2]
        k = wid + (t % 2) * NW
        pltpu.sync_copy(src.at[k], row_v)

        def extract(j, carry, apply_exp=(t >= 2), ext_v=ext_v):
            base = pl.multiple_of(j * L, L)
            iv = idx_v[pl.ds(base, L)]
            x = plsc.load_gather(row_v, [iv])
            if apply_exp:
                x = jnp.exp(x * 2.0)
            ext_v[pl.ds(base, L)] = x
            return carry

        lax.fori_loop(0, B // L, extract, 0, unroll=4)
        if len(wcps) >= 2:
            wcps[t - 2].wait()
        wcps.append(pltpu.async_copy(ext_v, dst.at[k], semw))

    for cp in hcps:
        cp.wait()
    for i in range(BPW // L):
        hv_v[pl.ds(i * L, L)] = jnp.exp(hv_v[pl.ds(i * L, L)] * 2.0)
    pltpu.sync_copy(hm_v, hm_hbm.at[pl.ds(base_b, BPW)])
    pltpu.sync_copy(hv_v, hv_hbm.at[pl.ds(base_b, BPW)])
    for cp in wcps[-2:]:
        cp.wait()


def kernel(indices, W_pos_mean, W_pos_logvar, W_het_mean, W_het_logvar):
    idx = indices.astype(jnp.int32)
    pm_t, pv_t, hm, hv = _sc_encoder(
        idx, W_pos_mean, W_pos_logvar, W_het_mean, W_het_logvar,
    )
    return (
        pm_t.T,
        pv_t.T,
        hm.reshape(B, 1),
        hv.reshape(B, 1),
    )
